# Initial kernel scaffold; baseline (speedup 1.0000x reference)
import jax
import jax.numpy as jnp
from jax.experimental import pallas as pl


def kernel(rule_goals, rule_success, queries, ent_emb, rel_emb):
    raise NotImplementedError("write your pallas kernel here")


# speedup vs baseline: 2.3621x; 2.3621x over previous
import functools

import jax
import jax.numpy as jnp
from jax import lax
from jax.experimental import pallas as pl
from jax.experimental.pallas import tpu as pltpu
from jax.experimental.pallas import tpu_sc as plsc

TOP_K = 32
CONSTANT_NO = 90000
PADDING_IDX = 0
DIM = 64

NC, NS, L = 2, 16, 16
NW = NC * NS
CHUNK = 128


def _sc_scores_body(p_hbm, a1_hbm, a2_hbm, succ_hbm, ent_hbm, rel_hbm, out_hbm,
                    p_v, a1_v, a2_v, succ_v, h_v, r_v, t_v, sc_v,
                    sem0, sem1, sem2):
    n = out_hbm.shape[0]
    t_per_w = n // NW
    n_chunks = t_per_w // CHUNK
    wid = lax.axis_index("s") * NC + lax.axis_index("c")
    wbase = wid * t_per_w

    def chunk_body(ci, _):
        base = wbase + ci * CHUNK
        pltpu.sync_copy(p_hbm.at[pl.ds(base, CHUNK)], p_v)
        pltpu.sync_copy(a1_hbm.at[pl.ds(base, CHUNK)], a1_v)
        pltpu.sync_copy(a2_hbm.at[pl.ds(base, CHUNK)], a2_v)
        pltpu.sync_copy(succ_hbm.at[pl.ds(base, CHUNK)], succ_v)
        c1 = pltpu.async_copy(ent_hbm.at[a1_v], h_v, sem0)
        c2 = pltpu.async_copy(rel_hbm.at[p_v], r_v, sem1)
        c3 = pltpu.async_copy(ent_hbm.at[a2_v], t_v, sem2)
        c1.wait()
        c2.wait()
        c3.wait()

        lane = lax.iota(jnp.int32, L)

        def group_body(g, _):
            rows = g * L + lane
            vals = jnp.zeros((L,), jnp.float32)
            for d in range(DIM):
                dvec = jnp.full((L,), d, jnp.int32)
                hh = plsc.load_gather(h_v, [rows, dvec])
                rr = plsc.load_gather(r_v, [rows, dvec])
                tt = plsc.load_gather(t_v, [rows, dvec])
                vals = vals + hh * rr * tt
            sl = pl.ds(g * L, L)
            ground = ((a1_v[sl] <= CONSTANT_NO) & (a2_v[sl] <= CONSTANT_NO)
                      & (p_v[sl] != PADDING_IDX))
            vals = jnp.where(ground, vals, jnp.zeros((L,), jnp.float32))
            vals = jnp.where(succ_v[sl] != 0, vals,
                             jnp.full((L,), -1e9, jnp.float32))
            sc_v[sl] = vals
            return ()

        lax.fori_loop(0, CHUNK // L, group_body, ())
        pltpu.sync_copy(sc_v, out_hbm.at[pl.ds(base, CHUNK)])
        return ()

    lax.fori_loop(0, n_chunks, chunk_body, ())


def _sc_scores(p, a1, a2, succ, ent_emb, rel_emb):
    n = p.shape[0]
    mesh = plsc.VectorSubcoreMesh(core_axis_name="c", subcore_axis_name="s",
                                  num_cores=NC, num_subcores=NS)
    return pl.kernel(
        _sc_scores_body,
        out_type=jax.ShapeDtypeStruct((n,), jnp.float32),
        mesh=mesh,
        compiler_params=pltpu.CompilerParams(needs_layout_passes=False,
                                             use_tc_tiling_on_sc=False),
        scratch_types=[
            pltpu.VMEM((CHUNK,), jnp.int32),
            pltpu.VMEM((CHUNK,), jnp.int32),
            pltpu.VMEM((CHUNK,), jnp.int32),
            pltpu.VMEM((CHUNK,), jnp.int32),
            pltpu.VMEM((CHUNK, DIM), jnp.float32),
            pltpu.VMEM((CHUNK, DIM), jnp.float32),
            pltpu.VMEM((CHUNK, DIM), jnp.float32),
            pltpu.VMEM((CHUNK,), jnp.float32),
            pltpu.SemaphoreType.DMA,
            pltpu.SemaphoreType.DMA,
            pltpu.SemaphoreType.DMA,
        ],
    )(p, a1, a2, succ, ent_emb, rel_emb)


def _tc_topk_body(s_ref, succ_ref, out_ref, s_scr, keep_scr):
    rb, kr = s_ref.shape
    col = lax.broadcasted_iota(jnp.int32, (rb, kr), 1)
    s_scr[...] = s_ref[...]
    keep_scr[...] = jnp.zeros((rb, kr), jnp.int32)

    def it(_, carry):
        s = s_scr[...]
        m = jnp.max(s, axis=1, keepdims=True)
        first_idx = jnp.min(jnp.where(s == m, col, kr), axis=1, keepdims=True)
        onehot = col == first_idx
        keep_scr[...] = keep_scr[...] | onehot.astype(jnp.int32)
        s_scr[...] = jnp.where(onehot, jnp.float32(-jnp.inf), s)
        return carry

    lax.fori_loop(0, TOP_K, it, 0)
    out_ref[...] = keep_scr[...] & (succ_ref[...] != 0).astype(jnp.int32)


def _tc_topk(scores2d, succ2d, interpret=False):
    n_rows, kr = scores2d.shape
    rb = 256
    return pl.pallas_call(
        _tc_topk_body,
        grid=(n_rows // rb,),
        in_specs=[pl.BlockSpec((rb, kr), lambda i: (i, 0)),
                  pl.BlockSpec((rb, kr), lambda i: (i, 0))],
        out_specs=pl.BlockSpec((rb, kr), lambda i: (i, 0)),
        out_shape=jax.ShapeDtypeStruct((n_rows, kr), jnp.int32),
        scratch_shapes=[pltpu.VMEM((rb, kr), jnp.float32),
                        pltpu.VMEM((rb, kr), jnp.int32)],
        interpret=interpret,
    )(scores2d, succ2d)


def kernel(rule_goals, rule_success, queries, ent_emb, rel_emb):
    b, s, kr = rule_success.shape
    first = rule_goals[:, :, :, 0, :].reshape(-1, 3)
    p = first[:, 0]
    a1 = first[:, 1]
    a2 = first[:, 2]
    succ = rule_success.reshape(-1).astype(jnp.int32)
    scores = _sc_scores(p, a1, a2, succ, ent_emb, rel_emb)
    keep = _tc_topk(scores.reshape(b * s, kr), succ.reshape(b * s, kr))
    return rule_success & (keep != 0).reshape(b, s, kr)


# speedup vs baseline: 2.7746x; 1.1746x over previous
import functools

import jax
import jax.numpy as jnp
from jax import lax
from jax.experimental import pallas as pl
from jax.experimental.pallas import tpu as pltpu
from jax.experimental.pallas import tpu_sc as plsc

TOP_K = 32
CONSTANT_NO = 90000
PADDING_IDX = 0
DIM = 64

NC, NS, L = 2, 16, 16
NW = NC * NS
CHUNK = 128


def _sc_scores_body(comb_hbm, ent_hbm, rel_hbm, out_hbm,
                    idx_v, ha_v, ra_v, ta_v, hb_v, rb_v, tb_v, sc_v,
                    sem_a, sem_b):
    t_per_w = comb_hbm.shape[1] // 4
    n_chunks = t_per_w // CHUNK
    wid = lax.axis_index("s") * NC + lax.axis_index("c")
    wbase = wid * t_per_w
    lane = lax.iota(jnp.int32, L)

    pltpu.sync_copy(comb_hbm.at[wid], idx_v)

    def gathers(c, h, r, t, sem):
        ip = idx_v.at[pl.ds(c * CHUNK, CHUNK)]
        ia1 = idx_v.at[pl.ds(t_per_w + c * CHUNK, CHUNK)]
        ia2 = idx_v.at[pl.ds(2 * t_per_w + c * CHUNK, CHUNK)]
        return (pltpu.make_async_copy(ent_hbm.at[ia1], h, sem),
                pltpu.make_async_copy(rel_hbm.at[ip], r, sem),
                pltpu.make_async_copy(ent_hbm.at[ia2], t, sem))

    def fire(c, h, r, t, sem):
        for cp in gathers(c, h, r, t, sem):
            cp.start()

    def drain(c, h, r, t, sem):
        for cp in gathers(c, h, r, t, sem):
            cp.wait()

    def compute(c, h, r, t):
        def group_body(g, _):
            rows = g * L + lane
            vals = jnp.zeros((L,), jnp.float32)
            for d in range(DIM):
                dvec = jnp.full((L,), d, jnp.int32)
                vals = vals + (plsc.load_gather(h, [rows, dvec])
                               * plsc.load_gather(r, [rows, dvec])
                               * plsc.load_gather(t, [rows, dvec]))
            o = c * CHUNK + g * L
            ground = ((idx_v[pl.ds(t_per_w + o, L)] <= CONSTANT_NO)
                      & (idx_v[pl.ds(2 * t_per_w + o, L)] <= CONSTANT_NO)
                      & (idx_v[pl.ds(o, L)] != PADDING_IDX))
            vals = jnp.where(ground, vals, jnp.zeros((L,), jnp.float32))
            vals = jnp.where(idx_v[pl.ds(3 * t_per_w + o, L)] != 0, vals,
                             jnp.full((L,), -1e9, jnp.float32))
            sc_v[pl.ds(o, L)] = vals
            return ()

        lax.fori_loop(0, CHUNK // L, group_body, ())

    fire(0, ha_v, ra_v, ta_v, sem_a)

    def pair_body(i, _):
        c0 = 2 * i
        fire(c0 + 1, hb_v, rb_v, tb_v, sem_b)
        drain(c0, ha_v, ra_v, ta_v, sem_a)
        compute(c0, ha_v, ra_v, ta_v)

        @pl.when(c0 + 2 < n_chunks)
        def _():
            fire(c0 + 2, ha_v, ra_v, ta_v, sem_a)

        drain(c0 + 1, hb_v, rb_v, tb_v, sem_b)
        compute(c0 + 1, hb_v, rb_v, tb_v)
        return ()

    lax.fori_loop(0, n_chunks // 2, pair_body, ())
    pltpu.sync_copy(sc_v, out_hbm.at[pl.ds(wbase, t_per_w)])


def _sc_scores(comb, ent_emb, rel_emb):
    n = comb.shape[0] * comb.shape[1] // 4
    t_per_w = n // NW
    mesh = plsc.VectorSubcoreMesh(core_axis_name="c", subcore_axis_name="s",
                                  num_cores=NC, num_subcores=NS)
    return pl.kernel(
        _sc_scores_body,
        out_type=jax.ShapeDtypeStruct((n,), jnp.float32),
        mesh=mesh,
        compiler_params=pltpu.CompilerParams(needs_layout_passes=False,
                                             use_tc_tiling_on_sc=False),
        scratch_types=[
            pltpu.VMEM((4 * t_per_w,), jnp.int32),
            pltpu.VMEM((CHUNK, DIM), jnp.float32),
            pltpu.VMEM((CHUNK, DIM), jnp.float32),
            pltpu.VMEM((CHUNK, DIM), jnp.float32),
            pltpu.VMEM((CHUNK, DIM), jnp.float32),
            pltpu.VMEM((CHUNK, DIM), jnp.float32),
            pltpu.VMEM((CHUNK, DIM), jnp.float32),
            pltpu.VMEM((t_per_w,), jnp.float32),
            pltpu.SemaphoreType.DMA,
            pltpu.SemaphoreType.DMA,
        ],
    )(comb, ent_emb, rel_emb)


def _tc_topk_body(s_ref, succ_ref, out_ref, s_scr, keep_scr):
    rb, kr = s_ref.shape
    col = lax.broadcasted_iota(jnp.int32, (rb, kr), 1)
    s_scr[...] = s_ref[...]
    keep_scr[...] = jnp.zeros((rb, kr), jnp.int32)

    def it(_, carry):
        s = s_scr[...]
        m = jnp.max(s, axis=1, keepdims=True)
        first_idx = jnp.min(jnp.where(s == m, col, kr), axis=1, keepdims=True)
        onehot = col == first_idx
        keep_scr[...] = keep_scr[...] | onehot.astype(jnp.int32)
        s_scr[...] = jnp.where(onehot, jnp.float32(-jnp.inf), s)
        return carry

    lax.fori_loop(0, TOP_K, it, 0)
    out_ref[...] = keep_scr[...] & (succ_ref[...] != 0).astype(jnp.int32)


def _tc_topk(scores2d, succ2d, interpret=False):
    n_rows, kr = scores2d.shape
    rb = 256
    return pl.pallas_call(
        _tc_topk_body,
        grid=(n_rows // rb,),
        in_specs=[pl.BlockSpec((rb, kr), lambda i: (i, 0)),
                  pl.BlockSpec((rb, kr), lambda i: (i, 0))],
        out_specs=pl.BlockSpec((rb, kr), lambda i: (i, 0)),
        out_shape=jax.ShapeDtypeStruct((n_rows, kr), jnp.int32),
        scratch_shapes=[pltpu.VMEM((rb, kr), jnp.float32),
                        pltpu.VMEM((rb, kr), jnp.int32)],
        interpret=interpret,
    )(scores2d, succ2d)


def kernel(rule_goals, rule_success, queries, ent_emb, rel_emb):
    b, s, kr = rule_success.shape
    n = b * s * kr
    t_per_w = n // NW
    first = rule_goals[:, :, :, 0, :].reshape(-1, 3)
    succ = rule_success.reshape(-1).astype(jnp.int32)
    comb = jnp.stack([first[:, 0].reshape(NW, t_per_w),
                      first[:, 1].reshape(NW, t_per_w),
                      first[:, 2].reshape(NW, t_per_w),
                      succ.reshape(NW, t_per_w)], axis=1).reshape(NW, 4 * t_per_w)
    scores = _sc_scores(comb, ent_emb, rel_emb)
    keep = _tc_topk(scores.reshape(b * s, kr), succ.reshape(b * s, kr))
    return rule_success & (keep != 0).reshape(b, s, kr)


# speedup vs baseline: 8.4616x; 3.0497x over previous
import functools

import jax
import jax.numpy as jnp
from jax import lax
from jax.experimental import pallas as pl
from jax.experimental.pallas import tpu as pltpu
from jax.experimental.pallas import tpu_sc as plsc

TOP_K = 32
CONSTANT_NO = 90000
PADDING_IDX = 0
DIM = 64

NC, NS, L = 2, 16, 16
NW = NC * NS
CHUNK = 128


def _sc_scores_body(comb_hbm, ent_hbm, rel_hbm, out_hbm,
                    idx_v, ha_v, ra_v, ta_v, hb_v, rb_v, tb_v, prod_v, sc_v,
                    sem_a, sem_b):
    t_per_w = comb_hbm.shape[1] // 4
    n_chunks = t_per_w // CHUNK
    wid = lax.axis_index("s") * NC + lax.axis_index("c")
    wbase = wid * t_per_w
    lane = lax.iota(jnp.int32, L)

    pltpu.sync_copy(comb_hbm.at[wid], idx_v)

    def gathers(c, h, r, t, sem):
        ip = idx_v.at[pl.ds(c * CHUNK, CHUNK)]
        ia1 = idx_v.at[pl.ds(t_per_w + c * CHUNK, CHUNK)]
        ia2 = idx_v.at[pl.ds(2 * t_per_w + c * CHUNK, CHUNK)]
        return (pltpu.make_async_copy(ent_hbm.at[ia1], h, sem),
                pltpu.make_async_copy(rel_hbm.at[ip], r, sem),
                pltpu.make_async_copy(ent_hbm.at[ia2], t, sem))

    def fire(c, h, r, t, sem):
        for cp in gathers(c, h, r, t, sem):
            cp.start()

    def drain(c, h, r, t, sem):
        for cp in gathers(c, h, r, t, sem):
            cp.wait()

    wrot = wid & (L - 1)

    def compute(c, h, r, t):
        def group_body(g, _):
            for i in range(L):
                row = g * L + i
                for q in range(DIM // L):
                    sl = pl.ds(q * L, L)
                    prod_v[pl.ds((row * DIM) + q * L, L)] = (
                        h[row, sl] * r[row, sl] * t[row, sl])
            rowv = (g * (L * DIM)) + (lane * DIM)
            dbase = lane + wrot
            acc = [jnp.zeros((L,), jnp.float32) for _ in range(4)]
            for d in range(DIM):
                fidx = rowv + ((dbase + d) & (DIM - 1))
                acc[d % 4] = acc[d % 4] + plsc.load_gather(prod_v, [fidx])
            vals = (acc[0] + acc[1]) + (acc[2] + acc[3])
            o = c * CHUNK + g * L
            ground = ((idx_v[pl.ds(t_per_w + o, L)] <= CONSTANT_NO)
                      & (idx_v[pl.ds(2 * t_per_w + o, L)] <= CONSTANT_NO)
                      & (idx_v[pl.ds(o, L)] != PADDING_IDX))
            vals = jnp.where(ground, vals, jnp.zeros((L,), jnp.float32))
            vals = jnp.where(idx_v[pl.ds(3 * t_per_w + o, L)] != 0, vals,
                             jnp.full((L,), -1e9, jnp.float32))
            sc_v[pl.ds(o, L)] = vals
            return ()

        lax.fori_loop(0, CHUNK // L, group_body, ())

    fire(0, ha_v, ra_v, ta_v, sem_a)

    def pair_body(i, _):
        c0 = 2 * i
        fire(c0 + 1, hb_v, rb_v, tb_v, sem_b)
        drain(c0, ha_v, ra_v, ta_v, sem_a)
        compute(c0, ha_v, ra_v, ta_v)

        @pl.when(c0 + 2 < n_chunks)
        def _():
            fire(c0 + 2, ha_v, ra_v, ta_v, sem_a)

        drain(c0 + 1, hb_v, rb_v, tb_v, sem_b)
        compute(c0 + 1, hb_v, rb_v, tb_v)
        return ()

    lax.fori_loop(0, n_chunks // 2, pair_body, ())
    pltpu.sync_copy(sc_v, out_hbm.at[pl.ds(wbase, t_per_w)])


def _sc_scores(comb, ent_emb, rel_emb):
    n = comb.shape[0] * comb.shape[1] // 4
    t_per_w = n // NW
    mesh = plsc.VectorSubcoreMesh(core_axis_name="c", subcore_axis_name="s",
                                  num_cores=NC, num_subcores=NS)
    return pl.kernel(
        _sc_scores_body,
        out_type=jax.ShapeDtypeStruct((n,), jnp.float32),
        mesh=mesh,
        compiler_params=pltpu.CompilerParams(needs_layout_passes=False,
                                             use_tc_tiling_on_sc=False),
        scratch_types=[
            pltpu.VMEM((4 * t_per_w,), jnp.int32),
            pltpu.VMEM((CHUNK, DIM), jnp.float32),
            pltpu.VMEM((CHUNK, DIM), jnp.float32),
            pltpu.VMEM((CHUNK, DIM), jnp.float32),
            pltpu.VMEM((CHUNK, DIM), jnp.float32),
            pltpu.VMEM((CHUNK, DIM), jnp.float32),
            pltpu.VMEM((CHUNK, DIM), jnp.float32),
            pltpu.VMEM((CHUNK * DIM,), jnp.float32),
            pltpu.VMEM((t_per_w,), jnp.float32),
            pltpu.SemaphoreType.DMA,
            pltpu.SemaphoreType.DMA,
        ],
    )(comb, ent_emb, rel_emb)


def _tc_topk_body(s_ref, succ_ref, out_ref, s_scr, keep_scr):
    rb, kr = s_ref.shape
    col = lax.broadcasted_iota(jnp.int32, (rb, kr), 1)
    s_scr[...] = s_ref[...]
    keep_scr[...] = jnp.zeros((rb, kr), jnp.int32)

    def it(_, carry):
        s = s_scr[...]
        m = jnp.max(s, axis=1, keepdims=True)
        first_idx = jnp.min(jnp.where(s == m, col, kr), axis=1, keepdims=True)
        onehot = col == first_idx
        keep_scr[...] = keep_scr[...] | onehot.astype(jnp.int32)
        s_scr[...] = jnp.where(onehot, jnp.float32(-jnp.inf), s)
        return carry

    lax.fori_loop(0, TOP_K, it, 0)
    out_ref[...] = keep_scr[...] & (succ_ref[...] != 0).astype(jnp.int32)


def _tc_topk(scores2d, succ2d, interpret=False):
    n_rows, kr = scores2d.shape
    rb = 256
    return pl.pallas_call(
        _tc_topk_body,
        grid=(n_rows // rb,),
        in_specs=[pl.BlockSpec((rb, kr), lambda i: (i, 0)),
                  pl.BlockSpec((rb, kr), lambda i: (i, 0))],
        out_specs=pl.BlockSpec((rb, kr), lambda i: (i, 0)),
        out_shape=jax.ShapeDtypeStruct((n_rows, kr), jnp.int32),
        scratch_shapes=[pltpu.VMEM((rb, kr), jnp.float32),
                        pltpu.VMEM((rb, kr), jnp.int32)],
        interpret=interpret,
    )(scores2d, succ2d)


def kernel(rule_goals, rule_success, queries, ent_emb, rel_emb):
    b, s, kr = rule_success.shape
    n = b * s * kr
    t_per_w = n // NW
    first = rule_goals[:, :, :, 0, :].reshape(-1, 3)
    succ = rule_success.reshape(-1).astype(jnp.int32)
    comb = jnp.stack([first[:, 0].reshape(NW, t_per_w),
                      first[:, 1].reshape(NW, t_per_w),
                      first[:, 2].reshape(NW, t_per_w),
                      succ.reshape(NW, t_per_w)], axis=1).reshape(NW, 4 * t_per_w)
    scores = _sc_scores(comb, ent_emb, rel_emb)
    keep = _tc_topk(scores.reshape(b * s, kr), succ.reshape(b * s, kr))
    return rule_success & (keep != 0).reshape(b, s, kr)
